# baseline (speedup 1.0000x reference)
import functools

import jax
import jax.numpy as jnp
from jax.experimental import pallas as pl
from jax.experimental.pallas import tpu as pltpu

N, L, D1, D2, C = 200000, 96, 128, 128, 2
BLOCK = 20096
NB = -(-N // BLOCK)


def _clam_kernel(ht_ref, wfct_ref, bfc_ref, wabt_ref, bab_ref,
                 wct_ref, bc_ref, wcls_ref, bcls_ref,
                 araw_ref, logits_ref, yprob_ref, yhat_ref,
                 acc_ref, m_ref, s_ref):
    i = pl.program_id(0)

    @pl.when(i == 0)
    def _init():
        acc_ref[...] = jnp.zeros_like(acc_ref)
        m_ref[0, 0] = -jnp.inf
        s_ref[0, 0] = 0.0

    cols_left = N - i * BLOCK
    lane_id = jax.lax.broadcasted_iota(jnp.int32, (1, BLOCK), 1)
    lane_valid = lane_id < cols_left

    xt = jnp.maximum(
        jnp.dot(wfct_ref[...], ht_ref[...], preferred_element_type=jnp.float32)
        + bfc_ref[...], 0.0)
    xt = jax.lax.cond(cols_left < BLOCK,
                      lambda v: jnp.where(lane_valid, v, 0.0),
                      lambda v: v, xt)
    zt = (jnp.dot(wabt_ref[...], xt, preferred_element_type=jnp.float32)
          + bab_ref[...])
    at = jnp.tanh(zt[:D2, :])
    bt = 0.5 * jnp.tanh(0.5 * zt[D2:, :]) + 0.5
    A = (jnp.dot(wct_ref[...], at * bt, preferred_element_type=jnp.float32)
         + bc_ref[...])
    araw_ref[...] = A

    m_old = m_ref[0, 0]
    m_new = jnp.maximum(m_old, jnp.max(jnp.where(lane_valid, A, -jnp.inf)))
    p = jnp.where(lane_valid, jnp.exp(A - m_new), 0.0)
    scale = jnp.exp(m_old - m_new)
    s_ref[0, 0] = s_ref[0, 0] * scale + jnp.sum(p)
    pacc = jax.lax.dot_general(xt, p, (((1,), (1,)), ((), ())),
                               preferred_element_type=jnp.float32)
    acc_ref[...] = acc_ref[...] * scale + pacc
    m_ref[0, 0] = m_new

    @pl.when(i == NB - 1)
    def _finish():
        Mt = acc_ref[...] / s_ref[0, 0]
        logits = (jax.lax.dot_general(Mt, wcls_ref[...],
                                      (((0,), (0,)), ((), ())),
                                      preferred_element_type=jnp.float32)
                  + bcls_ref[...])
        logits_ref[...] = logits
        e = jnp.exp(logits - jnp.max(logits))
        yprob_ref[...] = e / jnp.sum(e)
        yhat_ref[...] = jnp.where(logits[:, 1:] > logits[:, :1], 1, 0
                                  ).astype(jnp.int32)


@functools.partial(jax.jit)
def _run(h, W_fc, b_fc, Wa, ba, Wb, bb, Wc, bc, Wcls, bcls):
    full = lambda shape: pl.BlockSpec(shape, lambda i: tuple(0 for _ in shape))
    araw, logits, yprob, yhat = pl.pallas_call(
        _clam_kernel,
        grid=(NB,),
        in_specs=[
            pl.BlockSpec((L, BLOCK), lambda i: (0, i)),
            full((D1, L)),
            full((D1, 1)),
            full((2 * D2, D1)),
            full((2 * D2, 1)),
            full((1, D2)),
            full((1, 1)),
            full((D1, C)),
            full((1, C)),
        ],
        out_specs=[
            pl.BlockSpec((1, BLOCK), lambda i: (0, i)),
            full((1, C)),
            full((1, C)),
            full((1, 1)),
        ],
        out_shape=[
            jax.ShapeDtypeStruct((1, NB * BLOCK), jnp.float32),
            jax.ShapeDtypeStruct((1, C), jnp.float32),
            jax.ShapeDtypeStruct((1, C), jnp.float32),
            jax.ShapeDtypeStruct((1, 1), jnp.int32),
        ],
        scratch_shapes=[
            pltpu.VMEM((D1, 1), jnp.float32),
            pltpu.SMEM((1, 1), jnp.float32),
            pltpu.SMEM((1, 1), jnp.float32),
        ],
    )(h.T, W_fc.T, b_fc.reshape(D1, 1),
      jnp.concatenate([Wa, Wb], axis=1).T,
      jnp.concatenate([ba, bb]).reshape(2 * D2, 1),
      Wc.reshape(1, D2), bc.reshape(1, 1),
      Wcls, bcls.reshape(1, C))
    return logits, yprob, yhat, araw[:, :N]


def kernel(h, W_fc, b_fc, Wa, ba, Wb, bb, Wc, bc, Wcls, bcls):
    logits, yprob, yhat, araw = _run(h, W_fc, b_fc, Wa, ba, Wb, bb, Wc, bc,
                                     Wcls, bcls)
    return (logits, yprob, yhat, araw)


# speedup vs baseline: 1.4115x; 1.4115x over previous
import functools

import jax
import jax.numpy as jnp
from jax.experimental import pallas as pl
from jax.experimental.pallas import tpu as pltpu

N, L, D1, D2, C = 200000, 96, 128, 128, 2
BLOCK = 20096
NB = -(-N // BLOCK)


def _clam_kernel(ht_ref, wfct_ref, bfc_ref, wabt_ref, bab_ref,
                 wct_ref, bc_ref, wcls_ref, bcls_ref,
                 araw_ref, logits_ref, yprob_ref, yhat_ref,
                 acc_ref, m_ref, s_ref):
    i = pl.program_id(0)

    @pl.when(i == 0)
    def _init():
        acc_ref[...] = jnp.zeros_like(acc_ref)
        m_ref[0, 0] = -jnp.inf
        s_ref[0, 0] = 0.0

    cols_left = N - i * BLOCK
    lane_id = jax.lax.broadcasted_iota(jnp.int32, (1, BLOCK), 1)
    lane_valid = lane_id < cols_left

    ht = jnp.where(lane_valid, ht_ref[...], 0.0)
    xt = jnp.maximum(
        jnp.dot(wfct_ref[...], ht, preferred_element_type=jnp.float32)
        + bfc_ref[...], 0.0)
    zt = (jnp.dot(wabt_ref[...], xt, preferred_element_type=jnp.float32)
          + bab_ref[...])
    at = jnp.tanh(zt[:D2, :])
    bt = 0.5 * jnp.tanh(zt[D2:, :]) + 0.5
    A = (jnp.dot(wct_ref[...], at * bt, preferred_element_type=jnp.float32)
         + bc_ref[...])
    araw_ref[...] = A

    m_old = m_ref[0, 0]
    m_new = jnp.maximum(m_old, jnp.max(jnp.where(lane_valid, A, -jnp.inf)))
    p = jnp.where(lane_valid, jnp.exp(A - m_new), 0.0)
    scale = jnp.exp(m_old - m_new)
    s_ref[0, 0] = s_ref[0, 0] * scale + jnp.sum(p)
    pacc = jnp.dot(xt, p.T, preferred_element_type=jnp.float32)
    acc_ref[...] = acc_ref[...] * scale + pacc
    m_ref[0, 0] = m_new

    @pl.when(i == NB - 1)
    def _finish():
        Mt = acc_ref[...] / s_ref[0, 0]
        logits = (jax.lax.dot_general(Mt, wcls_ref[...],
                                      (((0,), (0,)), ((), ())),
                                      preferred_element_type=jnp.float32)
                  + bcls_ref[...])
        logits_ref[...] = logits
        e = jnp.exp(logits - jnp.max(logits))
        yprob_ref[...] = e / jnp.sum(e)
        yhat_ref[...] = jnp.where(logits[:, 1:] > logits[:, :1], 1, 0
                                  ).astype(jnp.int32)


@functools.partial(jax.jit)
def _run(h, W_fc, b_fc, Wa, ba, Wb, bb, Wc, bc, Wcls, bcls):
    full = lambda shape: pl.BlockSpec(shape, lambda i: tuple(0 for _ in shape))
    araw, logits, yprob, yhat = pl.pallas_call(
        _clam_kernel,
        grid=(NB,),
        in_specs=[
            pl.BlockSpec((L, BLOCK), lambda i: (0, i)),
            full((D1, L)),
            full((D1, 1)),
            full((2 * D2, D1)),
            full((2 * D2, 1)),
            full((1, D2)),
            full((1, 1)),
            full((D1, C)),
            full((1, C)),
        ],
        out_specs=[
            pl.BlockSpec((1, BLOCK), lambda i: (0, i)),
            full((1, C)),
            full((1, C)),
            full((1, 1)),
        ],
        out_shape=[
            jax.ShapeDtypeStruct((1, NB * BLOCK), jnp.float32),
            jax.ShapeDtypeStruct((1, C), jnp.float32),
            jax.ShapeDtypeStruct((1, C), jnp.float32),
            jax.ShapeDtypeStruct((1, 1), jnp.int32),
        ],
        scratch_shapes=[
            pltpu.VMEM((D1, 1), jnp.float32),
            pltpu.SMEM((1, 1), jnp.float32),
            pltpu.SMEM((1, 1), jnp.float32),
        ],
    )(h.T, W_fc.T, b_fc.reshape(D1, 1),
      jnp.concatenate([Wa, Wb * 0.5], axis=1).T,
      jnp.concatenate([ba, bb * 0.5]).reshape(2 * D2, 1),
      Wc.reshape(1, D2), bc.reshape(1, 1),
      Wcls, bcls.reshape(1, C))
    return logits, yprob, yhat, araw[:, :N]


def kernel(h, W_fc, b_fc, Wa, ba, Wb, bb, Wc, bc, Wcls, bcls):
    logits, yprob, yhat, araw = _run(h, W_fc, b_fc, Wa, ba, Wb, bb, Wc, bc,
                                     Wcls, bcls)
    return (logits, yprob, yhat, araw)


# speedup vs baseline: 1.5392x; 1.0904x over previous
import functools

import jax
import jax.numpy as jnp
from jax.experimental import pallas as pl
from jax.experimental.pallas import tpu as pltpu

N, L, D1, D2, C = 200000, 96, 128, 128, 2
BLOCK = 20096
NB = -(-N // BLOCK)


def _clam_kernel(ht_ref, wfct_ref, bfc_ref, wabt_ref, bab_ref,
                 wct_ref, bc_ref, wcls_ref, bcls_ref,
                 araw_ref, logits_ref, yprob_ref, yhat_ref,
                 acc_ref, m_ref, s_ref):
    i = pl.program_id(0)

    @pl.when(i == 0)
    def _init():
        acc_ref[...] = jnp.zeros_like(acc_ref)
        m_ref[0, 0] = -jnp.inf
        s_ref[0, 0] = 0.0

    cols_left = N - i * BLOCK
    lane_id = jax.lax.broadcasted_iota(jnp.int32, (1, BLOCK), 1)
    lane_valid = lane_id < cols_left

    ht = jnp.where(lane_valid, ht_ref[...], 0.0)
    xt = jnp.maximum(
        jnp.dot(wfct_ref[...], ht, preferred_element_type=jnp.float32)
        + bfc_ref[...], 0.0)
    zt = (jnp.dot(wabt_ref[...], xt, preferred_element_type=jnp.float32)
          + bab_ref[...])
    at = jnp.tanh(zt[:D2, :])
    bt = 0.5 * jnp.tanh(zt[D2:, :]) + 0.5
    A = (jnp.dot(wct_ref[...], at * bt, preferred_element_type=jnp.float32)
         + bc_ref[...])
    araw_ref[...] = A

    m_old = m_ref[0, 0]
    m_new = jnp.maximum(m_old, jnp.max(jnp.where(lane_valid, A, -jnp.inf)))
    p = jnp.where(lane_valid, jnp.exp(A - m_new), 0.0)
    scale = jnp.exp(m_old - m_new)
    s_ref[0, 0] = s_ref[0, 0] * scale + jnp.sum(p)
    pacc = jax.lax.dot_general(p, xt, (((1,), (1,)), ((), ())),
                               preferred_element_type=jnp.float32)
    acc_ref[...] = acc_ref[...] * scale + pacc
    m_ref[0, 0] = m_new

    @pl.when(i == NB - 1)
    def _finish():
        M = acc_ref[...] / s_ref[0, 0]
        logits = (jnp.dot(M, wcls_ref[...],
                          preferred_element_type=jnp.float32)
                  + bcls_ref[...])
        logits_ref[...] = logits
        e = jnp.exp(logits - jnp.max(logits))
        yprob_ref[...] = e / jnp.sum(e)
        yhat_ref[...] = jnp.where(logits[:, 1:] > logits[:, :1], 1, 0
                                  ).astype(jnp.int32)


@functools.partial(jax.jit)
def _run(h, W_fc, b_fc, Wa, ba, Wb, bb, Wc, bc, Wcls, bcls):
    full = lambda shape: pl.BlockSpec(shape, lambda i: tuple(0 for _ in shape))
    araw, logits, yprob, yhat = pl.pallas_call(
        _clam_kernel,
        grid=(NB,),
        in_specs=[
            pl.BlockSpec((L, BLOCK), lambda i: (0, i)),
            full((D1, L)),
            full((D1, 1)),
            full((2 * D2, D1)),
            full((2 * D2, 1)),
            full((1, D2)),
            full((1, 1)),
            full((D1, C)),
            full((1, C)),
        ],
        out_specs=[
            pl.BlockSpec((1, BLOCK), lambda i: (0, i)),
            full((1, C)),
            full((1, C)),
            full((1, 1)),
        ],
        out_shape=[
            jax.ShapeDtypeStruct((1, NB * BLOCK), jnp.float32),
            jax.ShapeDtypeStruct((1, C), jnp.float32),
            jax.ShapeDtypeStruct((1, C), jnp.float32),
            jax.ShapeDtypeStruct((1, 1), jnp.int32),
        ],
        scratch_shapes=[
            pltpu.VMEM((1, D1), jnp.float32),
            pltpu.SMEM((1, 1), jnp.float32),
            pltpu.SMEM((1, 1), jnp.float32),
        ],
    )(h.T, W_fc.T, b_fc.reshape(D1, 1),
      jnp.concatenate([Wa, Wb * 0.5], axis=1).T,
      jnp.concatenate([ba, bb * 0.5]).reshape(2 * D2, 1),
      Wc.reshape(1, D2), bc.reshape(1, 1),
      Wcls, bcls.reshape(1, C))
    return logits, yprob, yhat, araw[:, :N]


def kernel(h, W_fc, b_fc, Wa, ba, Wb, bb, Wc, bc, Wcls, bcls):
    logits, yprob, yhat, araw = _run(h, W_fc, b_fc, Wa, ba, Wb, bb, Wc, bc,
                                     Wcls, bcls)
    return (logits, yprob, yhat, araw)


# speedup vs baseline: 1.5502x; 1.0072x over previous
import functools

import jax
import jax.numpy as jnp
from jax.experimental import pallas as pl
from jax.experimental.pallas import tpu as pltpu

N, L, D1, D2, C = 200000, 96, 128, 128, 2
BLOCK = 20096
NB = -(-N // BLOCK)
LA = L + 8
D1A = D1 + 8


def _clam_kernel(ht_ref, wfcta_ref, wabta_ref, wct_ref, bc_ref,
                 wcls_ref, bcls_ref,
                 araw_ref, logits_ref, yprob_ref, yhat_ref,
                 haug_ref, xaug_ref, acc_ref, m_ref, s_ref):
    i = pl.program_id(0)

    @pl.when(i == 0)
    def _init():
        haug_ref[L:, :] = jnp.ones_like(haug_ref[L:, :])
        xaug_ref[D1:, :] = jnp.ones_like(xaug_ref[D1:, :])
        acc_ref[...] = jnp.zeros_like(acc_ref)
        m_ref[0, 0] = -jnp.inf
        s_ref[0, 0] = 0.0

    cols_left = N - i * BLOCK
    lane_id = jax.lax.broadcasted_iota(jnp.int32, (1, BLOCK), 1)
    lane_valid = lane_id < cols_left

    haug_ref[:L, :] = jnp.where(lane_valid, ht_ref[...], 0.0)
    xaug_ref[:D1, :] = jnp.maximum(
        jnp.dot(wfcta_ref[...], haug_ref[...],
                preferred_element_type=jnp.float32), 0.0)
    zt = jnp.dot(wabta_ref[...], xaug_ref[...],
                 preferred_element_type=jnp.float32)
    at = jnp.tanh(zt[:D2, :])
    bt = 0.5 * jnp.tanh(zt[D2:, :]) + 0.5
    A = (jnp.dot(wct_ref[...], at * bt, preferred_element_type=jnp.float32)
         + bc_ref[...])
    araw_ref[...] = A

    m_old = m_ref[0, 0]
    m_new = jnp.maximum(m_old, jnp.max(jnp.where(lane_valid, A, -jnp.inf)))
    p = jnp.where(lane_valid, jnp.exp(A - m_new), 0.0)
    scale = jnp.exp(m_old - m_new)
    s_ref[0, 0] = s_ref[0, 0] * scale + jnp.sum(p)
    pacc = jax.lax.dot_general(p, xaug_ref[:D1, :], (((1,), (1,)), ((), ())),
                               preferred_element_type=jnp.float32)
    acc_ref[...] = acc_ref[...] * scale + pacc
    m_ref[0, 0] = m_new

    @pl.when(i == NB - 1)
    def _finish():
        M = acc_ref[...] / s_ref[0, 0]
        logits = (jnp.dot(M, wcls_ref[...],
                          preferred_element_type=jnp.float32)
                  + bcls_ref[...])
        logits_ref[...] = logits
        e = jnp.exp(logits - jnp.max(logits))
        yprob_ref[...] = e / jnp.sum(e)
        yhat_ref[...] = jnp.where(logits[:, 1:] > logits[:, :1], 1, 0
                                  ).astype(jnp.int32)


@functools.partial(jax.jit)
def _run(h, W_fc, b_fc, Wa, ba, Wb, bb, Wc, bc, Wcls, bcls):
    full = lambda shape: pl.BlockSpec(shape, lambda i: tuple(0 for _ in shape))
    wfcta = jnp.concatenate(
        [W_fc.T, b_fc.reshape(D1, 1), jnp.zeros((D1, LA - L - 1), jnp.float32)],
        axis=1)
    wabt = jnp.concatenate([Wa, Wb * 0.5], axis=1).T
    bab = jnp.concatenate([ba, bb * 0.5]).reshape(2 * D2, 1)
    wabta = jnp.concatenate(
        [wabt, bab, jnp.zeros((2 * D2, D1A - D1 - 1), jnp.float32)],
        axis=1)
    araw, logits, yprob, yhat = pl.pallas_call(
        _clam_kernel,
        grid=(NB,),
        in_specs=[
            pl.BlockSpec((L, BLOCK), lambda i: (0, i)),
            full((D1, LA)),
            full((2 * D2, D1A)),
            full((1, D2)),
            full((1, 1)),
            full((D1, C)),
            full((1, C)),
        ],
        out_specs=[
            pl.BlockSpec((1, BLOCK), lambda i: (0, i)),
            full((1, C)),
            full((1, C)),
            full((1, 1)),
        ],
        out_shape=[
            jax.ShapeDtypeStruct((1, NB * BLOCK), jnp.float32),
            jax.ShapeDtypeStruct((1, C), jnp.float32),
            jax.ShapeDtypeStruct((1, C), jnp.float32),
            jax.ShapeDtypeStruct((1, 1), jnp.int32),
        ],
        scratch_shapes=[
            pltpu.VMEM((LA, BLOCK), jnp.float32),
            pltpu.VMEM((D1A, BLOCK), jnp.float32),
            pltpu.VMEM((1, D1), jnp.float32),
            pltpu.SMEM((1, 1), jnp.float32),
            pltpu.SMEM((1, 1), jnp.float32),
        ],
    )(h.T, wfcta, wabta, Wc.reshape(1, D2), bc.reshape(1, 1),
      Wcls, bcls.reshape(1, C))
    return logits, yprob, yhat, araw[:, :N]


def kernel(h, W_fc, b_fc, Wa, ba, Wb, bb, Wc, bc, Wcls, bcls):
    logits, yprob, yhat, araw = _run(h, W_fc, b_fc, Wa, ba, Wb, bb, Wc, bc,
                                     Wcls, bcls)
    return (logits, yprob, yhat, araw)


# speedup vs baseline: 1.5697x; 1.0125x over previous
import functools

import jax
import jax.numpy as jnp
from jax.experimental import pallas as pl
from jax.experimental.pallas import tpu as pltpu

N, L, D1, D2, C = 200000, 96, 128, 128, 2
BLOCK = 25088
NB = -(-N // BLOCK)
LA = L + 8
D1A = D1 + 8


def _clam_kernel(ht_ref, wfcta_ref, wabta_ref, wct_ref, bc_ref,
                 wcls_ref, bcls_ref,
                 araw_ref, logits_ref, yprob_ref, yhat_ref,
                 haug_ref, xaug_ref, acc_ref, m_ref, s_ref):
    i = pl.program_id(0)

    @pl.when(i == 0)
    def _init():
        haug_ref[L:, :] = jnp.ones_like(haug_ref[L:, :])
        xaug_ref[D1:, :] = jnp.ones_like(xaug_ref[D1:, :])
        acc_ref[...] = jnp.zeros_like(acc_ref)
        m_ref[0, 0] = -jnp.inf
        s_ref[0, 0] = 0.0

    cols_left = N - i * BLOCK
    lane_id = jax.lax.broadcasted_iota(jnp.int32, (1, BLOCK), 1)
    lane_valid = lane_id < cols_left

    haug_ref[:L, :] = jnp.where(lane_valid, ht_ref[...], 0.0)
    xaug_ref[:D1, :] = jnp.maximum(
        jnp.dot(wfcta_ref[...], haug_ref[...],
                preferred_element_type=jnp.float32), 0.0)
    zt = jnp.dot(wabta_ref[...], xaug_ref[...],
                 preferred_element_type=jnp.float32)
    at = jnp.tanh(zt[:D2, :])
    bt = 0.5 * jnp.tanh(zt[D2:, :]) + 0.5
    A = (jnp.dot(wct_ref[...], at * bt, preferred_element_type=jnp.float32)
         + bc_ref[...])
    araw_ref[...] = A

    m_old = m_ref[0, 0]
    m_new = jnp.maximum(m_old, jnp.max(jnp.where(lane_valid, A, -jnp.inf)))
    p = jnp.where(lane_valid, jnp.exp(A - m_new), 0.0)
    scale = jnp.exp(m_old - m_new)
    s_ref[0, 0] = s_ref[0, 0] * scale + jnp.sum(p)
    pacc = jax.lax.dot_general(p, xaug_ref[:D1, :], (((1,), (1,)), ((), ())),
                               preferred_element_type=jnp.float32)
    acc_ref[...] = acc_ref[...] * scale + pacc
    m_ref[0, 0] = m_new

    @pl.when(i == NB - 1)
    def _finish():
        M = acc_ref[...] / s_ref[0, 0]
        logits = (jnp.dot(M, wcls_ref[...],
                          preferred_element_type=jnp.float32)
                  + bcls_ref[...])
        logits_ref[...] = logits
        e = jnp.exp(logits - jnp.max(logits))
        yprob_ref[...] = e / jnp.sum(e)
        yhat_ref[...] = jnp.where(logits[:, 1:] > logits[:, :1], 1, 0
                                  ).astype(jnp.int32)


@functools.partial(jax.jit)
def _run(h, W_fc, b_fc, Wa, ba, Wb, bb, Wc, bc, Wcls, bcls):
    full = lambda shape: pl.BlockSpec(shape, lambda i: tuple(0 for _ in shape))
    wfcta = jnp.concatenate(
        [W_fc.T, b_fc.reshape(D1, 1), jnp.zeros((D1, LA - L - 1), jnp.float32)],
        axis=1)
    wabt = jnp.concatenate([Wa, Wb * 0.5], axis=1).T
    bab = jnp.concatenate([ba, bb * 0.5]).reshape(2 * D2, 1)
    wabta = jnp.concatenate(
        [wabt, bab, jnp.zeros((2 * D2, D1A - D1 - 1), jnp.float32)],
        axis=1)
    araw, logits, yprob, yhat = pl.pallas_call(
        _clam_kernel,
        grid=(NB,),
        in_specs=[
            pl.BlockSpec((L, BLOCK), lambda i: (0, i)),
            full((D1, LA)),
            full((2 * D2, D1A)),
            full((1, D2)),
            full((1, 1)),
            full((D1, C)),
            full((1, C)),
        ],
        out_specs=[
            pl.BlockSpec((1, BLOCK), lambda i: (0, i)),
            full((1, C)),
            full((1, C)),
            full((1, 1)),
        ],
        out_shape=[
            jax.ShapeDtypeStruct((1, NB * BLOCK), jnp.float32),
            jax.ShapeDtypeStruct((1, C), jnp.float32),
            jax.ShapeDtypeStruct((1, C), jnp.float32),
            jax.ShapeDtypeStruct((1, 1), jnp.int32),
        ],
        scratch_shapes=[
            pltpu.VMEM((LA, BLOCK), jnp.float32),
            pltpu.VMEM((D1A, BLOCK), jnp.float32),
            pltpu.VMEM((1, D1), jnp.float32),
            pltpu.SMEM((1, 1), jnp.float32),
            pltpu.SMEM((1, 1), jnp.float32),
        ],
    )(h.T, wfcta, wabta, Wc.reshape(1, D2), bc.reshape(1, 1),
      Wcls, bcls.reshape(1, C))
    return logits, yprob, yhat, araw[:, :N]


def kernel(h, W_fc, b_fc, Wa, ba, Wb, bb, Wc, bc, Wcls, bcls):
    logits, yprob, yhat, araw = _run(h, W_fc, b_fc, Wa, ba, Wb, bb, Wc, bc,
                                     Wcls, bcls)
    return (logits, yprob, yhat, araw)
